# baseline (speedup 1.0000x reference)
import jax
import jax.numpy as jnp
from jax import lax
from jax.experimental import pallas as pl
from jax.experimental.pallas import tpu as pltpu
from jax.experimental.pallas import tpu_sc as plsc

BATCH = 16384
EMB_DIM = 64
ROW_PITCH = 128
NUM_CORES = 2
NUM_SUBCORES = 16
NUM_WORKERS = NUM_CORES * NUM_SUBCORES
B_PER_W = BATCH // NUM_WORKERS
CHUNK = 128
N_CHUNKS = B_PER_W // CHUNK
GROUP = 16
N_SLOTS = 2


def _ranker_body(x1_hbm, x2_hbm, ue_hbm, ce_hbm, out_hbm,
                 idx1_v, idx2_v, p1_v, p2_v, u_b, c_b, out_v,
                 sem_i, sem0, sem1):
    sems = (sem0, sem1)
    wid = lax.axis_index("s") * NUM_CORES + lax.axis_index("c")
    base = wid * B_PER_W

    cp1 = pltpu.async_copy(x1_hbm.at[pl.ds(base, B_PER_W)], idx1_v, sem_i)
    cp2 = pltpu.async_copy(x2_hbm.at[pl.ds(base, B_PER_W)], idx2_v, sem_i)
    cp1.wait()
    cp2.wait()

    for k in range(B_PER_W // GROUP):
        sl = pl.ds(k * GROUP, GROUP)
        p1_v[sl] = lax.shift_right_logical(idx1_v[sl], 1)
        p2_v[sl] = lax.shift_right_logical(idx2_v[sl], 1)

    lane = lax.iota(jnp.int32, GROUP)

    def fire(q, s):
        iq = pl.ds(q * CHUNK, CHUNK)
        return (pltpu.async_copy(ue_hbm.at[p1_v.at[iq]], u_b.at[s], sems[s]),
                pltpu.async_copy(ce_hbm.at[p2_v.at[iq]], c_b.at[s], sems[s]))

    def compute(q, s):
        for g in range(CHUNK // GROUP):
            sl = pl.ds(q * CHUNK + g * GROUP, GROUP)
            h1 = jnp.bitwise_and(idx1_v[sl], 1) * EMB_DIM
            h2 = jnp.bitwise_and(idx2_v[sl], 1) * EMB_DIM
            res = jnp.zeros((GROUP,), jnp.float32)
            for j in range(GROUP):
                row = g * GROUP + j
                a = h1[j]
                b = h2[j]
                acc = (u_b[s, row, pl.ds(a, 16)]
                       * c_b[s, row, pl.ds(b, 16)])
                for k in range(1, EMB_DIM // 16):
                    acc = acc + (u_b[s, row, pl.ds(a + k * 16, 16)]
                                 * c_b[s, row, pl.ds(b + k * 16, 16)])
                res = jnp.where(lane == j, jnp.sum(acc), res)
            out_v[pl.ds(q * CHUNK + g * GROUP, GROUP)] = (
                1.0 / (1.0 + jnp.exp(-res)))

    pend = {0: fire(0, 0), 1: fire(1, 1)}
    for q in range(N_CHUNKS):
        s = q % N_SLOTS
        for cp in pend.pop(q):
            cp.wait()
        compute(q, s)
        if q + N_SLOTS < N_CHUNKS:
            pend[q + N_SLOTS] = fire(q + N_SLOTS, s)

    pltpu.sync_copy(out_v, out_hbm.at[pl.ds(base, B_PER_W)])


@jax.jit
def _ranker(x1, x2, ue, ce):
    n_u = ue.shape[0]
    n_c = ce.shape[0]
    ue2 = ue.reshape(n_u // 2, ROW_PITCH)
    ce2 = ce.reshape(n_c // 2, ROW_PITCH)
    mesh = plsc.VectorSubcoreMesh(core_axis_name="c", subcore_axis_name="s")
    return pl.kernel(
        _ranker_body,
        out_type=jax.ShapeDtypeStruct((BATCH,), jnp.float32),
        mesh=mesh,
        scratch_types=[
            pltpu.VMEM((B_PER_W,), jnp.int32),
            pltpu.VMEM((B_PER_W,), jnp.int32),
            pltpu.VMEM((B_PER_W,), jnp.int32),
            pltpu.VMEM((B_PER_W,), jnp.int32),
            pltpu.VMEM((N_SLOTS, CHUNK, ROW_PITCH), jnp.float32),
            pltpu.VMEM((N_SLOTS, CHUNK, ROW_PITCH), jnp.float32),
            pltpu.VMEM((B_PER_W,), jnp.float32),
            pltpu.SemaphoreType.DMA,
            pltpu.SemaphoreType.DMA,
            pltpu.SemaphoreType.DMA,
        ],
        compiler_params=pltpu.CompilerParams(needs_layout_passes=False),
    )(x1, x2, ue2, ce2)


def kernel(x1, x2, Ue, Ce, W, b):
    del W, b
    return _ranker(x1, x2, Ue, Ce)
